# Initial kernel scaffold; baseline (speedup 1.0000x reference)
import jax
import jax.numpy as jnp
from jax.experimental import pallas as pl


def kernel(left_input, right_input, mul_table, add_tables, final_table):
    raise NotImplementedError("write your pallas kernel here")


# speedup vs baseline: 107.9779x; 107.9779x over previous
import jax
import jax.numpy as jnp
from jax.experimental import pallas as pl
from jax.experimental.pallas import tpu as pltpu

B, I, M, O = 16, 256, 64, 256
KP = M // 2
NV = (I // 8) * (O // 128)


def _select_tree(cands, sel):
    h = 0
    while len(cands) > 1:
        m = ((sel >> h) & 1) == 1
        cands = [jnp.where(m, cands[2 * i + 1], cands[2 * i])
                 for i in range(len(cands) // 2)]
        h += 1
    return cands[0]


def _lut_byte(add_ref, level, t16):
    l = (t16 >> 2) & 127
    c = t16 >> 9
    sh = (t16 & 3) << 3
    cands = [jnp.take_along_axis(add_ref[level, ci], l, axis=1)
             for ci in range(128)]
    res = _select_tree(cands, c)
    return (res >> sh) & 255


def _lut_final(fin_ref, t16):
    l = (t16 >> 1) & 127
    c = t16 >> 8
    up = (1 - (t16 & 1)) << 4
    cands = [jnp.take_along_axis(fin_ref[ci], l, axis=1)
             for ci in range(256)]
    res = _select_tree(cands, c)
    bits = (res << up) & jnp.int32(-65536)
    return jax.lax.bitcast_convert_type(bits, jnp.float32)


def _plane_slice(v):
    r = pl.multiple_of((v >> 1) * 8, 8)
    c = pl.multiple_of((v & 1) * 128, 128)
    return pl.ds(r, 8), pl.ds(c, 128)


def _kernel(l_ref, r_ref, mul_ref, add_ref, fin_ref, out_ref, scr):
    kp = pl.program_id(1)

    lpair = l_ref[0, 0]
    rpair = r_ref[0, 0]
    lane_iota = jax.lax.broadcasted_iota(jnp.int32, (I, 256), 1)
    sub_iota = jax.lax.broadcasted_iota(jnp.int32, (256, O), 0)
    planes = []
    for rr in range(2):
        oh_l = (lpair[:, rr : rr + 1] == lane_iota).astype(jnp.bfloat16)
        oh_rt = (sub_iota == rpair[rr : rr + 1, :]).astype(jnp.bfloat16)
        rows = jnp.dot(oh_l, mul_ref[...], preferred_element_type=jnp.float32)
        p = jnp.dot(rows.astype(jnp.bfloat16), oh_rt,
                    preferred_element_type=jnp.float32)
        planes.append(p.astype(jnp.int32))
    scr[kp] = (planes[0] << 8) | planes[1]

    @pl.when(kp == KP - 1)
    def _epilogue():
        def l0_body(v, _):
            p = v >> 6
            rs, cs = _plane_slice(v & 63)
            scr[p, rs, cs] = _lut_byte(add_ref, 0, scr[p, rs, cs])
            return ()

        jax.lax.fori_loop(0, KP * NV, l0_body, ())

        for lvl in range(1, 5):
            n_out = KP >> lvl

            def lvl_body(v, _, lvl=lvl):
                p = v >> 6
                rs, cs = _plane_slice(v & 63)
                t16 = (scr[2 * p, rs, cs] << 8) | scr[2 * p + 1, rs, cs]
                scr[p, rs, cs] = _lut_byte(add_ref, lvl, t16)
                return ()

            jax.lax.fori_loop(0, n_out * NV, lvl_body, ())

        def fin_body(v, _):
            rs, cs = _plane_slice(v)
            t16 = (scr[0, rs, cs] << 8) | scr[1, rs, cs]
            out_ref[0, rs, cs] = _lut_final(fin_ref, t16)
            return ()

        jax.lax.fori_loop(0, NV, fin_body, ())


def _pack_add_tables(add_tables):
    a = add_tables.astype(jnp.int32)
    w = (a[:, :, 0::4] | (a[:, :, 1::4] << 8)
         | (a[:, :, 2::4] << 16) | (a[:, :, 3::4] << 24))
    w = w.reshape(5, 128, 1, 128)
    return jnp.broadcast_to(w, (5, 128, 8, 128))


def _pack_final_table(final_table):
    fb = jax.lax.bitcast_convert_type(
        final_table.astype(jnp.bfloat16), jnp.uint16).astype(jnp.int32)
    w = (fb[:, 0::2] | (fb[:, 1::2] << 16)).reshape(256, 1, 128)
    return jnp.broadcast_to(w, (256, 8, 128))


def kernel(left_input, right_input, mul_table, add_tables, final_table):
    left_r = jnp.transpose(left_input, (0, 2, 1)).reshape(B, KP, 2, I)
    left_r = jnp.transpose(left_r, (0, 1, 3, 2))
    right_r = right_input.reshape(B, KP, 2, O)
    mul_bf16 = mul_table.astype(jnp.bfloat16)
    add_packed = _pack_add_tables(add_tables)
    fin_packed = _pack_final_table(final_table)

    return pl.pallas_call(
        _kernel,
        grid=(B, KP),
        in_specs=[
            pl.BlockSpec((1, 1, I, 2), lambda b, k: (b, k, 0, 0)),
            pl.BlockSpec((1, 1, 2, O), lambda b, k: (b, k, 0, 0)),
            pl.BlockSpec((256, 256), lambda b, k: (0, 0)),
            pl.BlockSpec((5, 128, 8, 128), lambda b, k: (0, 0, 0, 0)),
            pl.BlockSpec((256, 8, 128), lambda b, k: (0, 0, 0)),
        ],
        out_specs=pl.BlockSpec((1, I, O), lambda b, k: (b, 0, 0)),
        out_shape=jax.ShapeDtypeStruct((B, I, O), jnp.float32),
        scratch_shapes=[pltpu.VMEM((KP, I, O), jnp.int32)],
        compiler_params=pltpu.CompilerParams(
            dimension_semantics=("parallel", "arbitrary"),
        ),
    )(left_r, right_r, mul_bf16, add_packed, fin_packed)
